# baseline (speedup 1.0000x reference)
import functools

import jax
import jax.numpy as jnp
from jax import lax
from jax.experimental import pallas as pl
from jax.experimental.pallas import tpu as pltpu
from jax.experimental.pallas import tpu_sc as plsc

_THRESHOLD = 0.5
_SC_CHUNK = 384


def _sc_extract(x_hbm, out_hbm, b0, b1, vals_v, s0, s1, *, n_per_w, nc, chan):
    wid = lax.axis_index("s") * nc + lax.axis_index("c")
    base = wid * n_per_w
    bufs = (b0, b1)
    sems = (s0, s1)
    n_chunks = n_per_w // _SC_CHUNK

    def _dma(k):
        return pltpu.make_async_copy(
            x_hbm.at[pl.ds(base + k * _SC_CHUNK, _SC_CHUNK), :],
            bufs[k % 2], sems[k % 2])

    lane = lax.broadcasted_iota(jnp.int32, (16,), 0)
    idx_chan = jnp.full((16,), chan, jnp.int32)

    _dma(0).start()
    for k in range(n_chunks):
        if k + 1 < n_chunks:
            _dma(k + 1).start()
        _dma(k).wait()
        buf = bufs[k % 2]

        def _grp(g, _, buf=buf, k=k):
            row0 = pl.multiple_of(g * 16, 16)
            acc = jnp.zeros((16,), jnp.float32)
            for j in range(16):
                v = buf[row0 + j, pl.ds(0, 16)]
                b = v.at[idx_chan].get(mode="promise_in_bounds")
                acc = jnp.where(lane == j, b, acc)
            vals_v[pl.ds(k * _SC_CHUNK + row0, 16)] = acc
            return 0

        lax.fori_loop(0, _SC_CHUNK // 16, _grp, 0, unroll=False)
    pltpu.sync_copy(vals_v, out_hbm.at[pl.ds(base, n_per_w)])


def _sc_gather_channel(x2d, chan):
    n_rows, C = x2d.shape
    info = plsc.get_sparse_core_info()
    nc, ns = info.num_cores, info.num_subcores
    n_per_w = n_rows // (nc * ns)
    mesh = plsc.VectorSubcoreMesh(core_axis_name="c", subcore_axis_name="s")
    return pl.kernel(
        functools.partial(_sc_extract, n_per_w=n_per_w, nc=nc, chan=chan),
        mesh=mesh,
        out_type=jax.ShapeDtypeStruct((n_rows,), jnp.float32),
        scratch_types=[
            pltpu.VMEM((_SC_CHUNK, C), jnp.float32),
            pltpu.VMEM((_SC_CHUNK, C), jnp.float32),
            pltpu.VMEM((n_per_w,), jnp.float32),
            pltpu.SemaphoreType.DMA,
            pltpu.SemaphoreType.DMA,
        ],
    )(x2d)


def _tc_body(t_ref, an_ref, p_ref, out_ref, *, A, H, W, T_total, T_per_b,
             n_elems, n_rows, n_lanes):
    pred = p_ref[:, :]

    tx = t_ref[1:2, :]
    ty = t_ref[2:3, :]
    tw = t_ref[3:4, :] * float(W)
    th = t_ref[4:5, :] * float(H)
    area_t = tw * th

    best_iou = None
    best_a = jnp.zeros_like(tx, dtype=jnp.int32)
    for k in range(A):
        aw = an_ref[k:k + 1, 0:1]
        ah = an_ref[k:k + 1, 1:2]
        inter = jnp.minimum(aw, tw) * jnp.minimum(ah, th)
        iou = inter / (aw * ah + area_t - inter)
        if k == 0:
            best_iou = iou
        else:
            upd = iou > best_iou
            best_a = jnp.where(upd, k, best_a)
            best_iou = jnp.where(upd, iou, best_iou)

    t_i = (tx * float(W)).astype(jnp.int32)
    t_j = (ty * float(H)).astype(jnp.int32)
    t_b = jax.lax.broadcasted_iota(jnp.int32, tx.shape, 1) // T_per_b
    hit = (best_iou > _THRESHOLD).astype(jnp.float32)

    lin = ((t_b * A + best_a) * H + t_j) * W + t_i
    r_u = jax.lax.div(lin, n_lanes)
    l_u = jax.lax.rem(lin, n_lanes)

    row_iota = jax.lax.broadcasted_iota(jnp.int32, (n_rows, T_total), 0)
    lane_iota = jax.lax.broadcasted_iota(jnp.int32, (n_lanes, T_total), 0)
    rh = (row_iota == r_u).astype(jnp.float32) * hit
    lh = (lane_iota == l_u).astype(jnp.float32)
    cnt = jax.lax.dot_general(rh, lh, (((1,), (1,)), ((), ())),
                              preferred_element_type=jnp.float32)
    gt = cnt > 0.0

    log_p = jnp.maximum(jnp.log(pred), -100.0)
    log_1p = jnp.maximum(jnp.log(1.0 - pred), -100.0)
    s_sum = jnp.sum(jnp.where(gt, -log_p, -log_1p))
    out_ref[0, 0] = s_sum / float(n_elems)


def kernel(output, anchors, targets):
    B, A, H, W, C = output.shape
    T = targets.shape[1]
    n_elems = B * A * H * W
    n_lanes = 128
    n_rows = n_elems // n_lanes

    pred_flat = _sc_gather_channel(output.reshape(n_elems, C), 4)
    pred = pred_flat.reshape(n_rows, n_lanes)
    tt = targets.reshape(B * T, 5).T

    out = pl.pallas_call(
        functools.partial(_tc_body, A=A, H=H, W=W, T_total=B * T, T_per_b=T,
                          n_elems=n_elems, n_rows=n_rows, n_lanes=n_lanes),
        in_specs=[
            pl.BlockSpec((5, B * T), lambda: (0, 0)),
            pl.BlockSpec((A, 2), lambda: (0, 0)),
            pl.BlockSpec((n_rows, n_lanes), lambda: (0, 0)),
        ],
        out_specs=pl.BlockSpec(memory_space=pltpu.SMEM),
        out_shape=jax.ShapeDtypeStruct((1, 1), jnp.float32),
    )(tt, anchors, pred)
    return out[0, 0]


# speedup vs baseline: 1.2227x; 1.2227x over previous
import functools

import jax
import jax.numpy as jnp
from jax import lax
from jax.experimental import pallas as pl
from jax.experimental.pallas import tpu as pltpu
from jax.experimental.pallas import tpu_sc as plsc

_THRESHOLD = 0.5
_NA_PLANES = 24
_BS = 8
_SC_CHUNK = 384


def _match_targets(t_ref, an_ref, A, H, W, T_per_b):
    tx = t_ref[1:2, :]
    ty = t_ref[2:3, :]
    tw = t_ref[3:4, :] * float(W)
    th = t_ref[4:5, :] * float(H)
    area_t = tw * th

    best_iou = None
    best_a = jnp.zeros_like(tx, dtype=jnp.int32)
    for k in range(A):
        aw = an_ref[k:k + 1, 0:1]
        ah = an_ref[k:k + 1, 1:2]
        inter = jnp.minimum(aw, tw) * jnp.minimum(ah, th)
        iou = inter / (aw * ah + area_t - inter)
        if k == 0:
            best_iou = iou
        else:
            upd = iou > best_iou
            best_a = jnp.where(upd, k, best_a)
            best_iou = jnp.where(upd, iou, best_iou)

    t_i = (tx * float(W)).astype(jnp.int32)
    t_j = (ty * float(H)).astype(jnp.int32)
    t_b = jax.lax.broadcasted_iota(jnp.int32, tx.shape, 1) // T_per_b
    hit = (best_iou > _THRESHOLD).astype(jnp.float32)
    lin = ((t_b * A + best_a) * H + t_j) * W + t_i
    return hit, lin


def _bce_sum(pred, gt):
    log_p = jnp.maximum(jnp.log(pred), -100.0)
    log_1p = jnp.maximum(jnp.log(1.0 - pred), -100.0)
    return jnp.sum(jnp.where(gt, -log_p, -log_1p))


def _tc_a_body(t_ref, an_ref, x_ref, out_ref, *, A, H, W, T_total, T_per_b):
    i = pl.program_id(0)
    n = pl.num_programs(0)
    pred = x_ref[:, :, :, 4].reshape(_BS * H, W)

    hit, lin = _match_targets(t_ref, an_ref, A, H, W, T_per_b)
    plane_u = jax.lax.div(lin, H * W)
    cell_u = jax.lax.rem(lin, H * W)
    t_j = jax.lax.div(cell_u, W)
    t_i = jax.lax.rem(cell_u, W)

    row_iota = jax.lax.broadcasted_iota(jnp.int32, (H, T_total), 0)
    col_iota = jax.lax.broadcasted_iota(jnp.int32, (W, T_total), 0)
    oj_base = (row_iota == t_j)
    oi = (col_iota == t_i).astype(jnp.float32)

    oj_rows = []
    for s in range(_BS):
        plane = i * _BS + s
        sel = (hit * (plane_u == plane).astype(jnp.float32))
        oj_rows.append(oj_base.astype(jnp.float32) * sel)
    oj = jnp.concatenate(oj_rows, axis=0)
    cnt = jax.lax.dot_general(oj, oi, (((1,), (1,)), ((), ())),
                              preferred_element_type=jnp.float32)
    gt = cnt > 0.0

    s_sum = _bce_sum(pred, gt)
    acc = jnp.where(i == 0, 0.0, out_ref[0, 0]) + s_sum
    out_ref[0, 0] = acc


def _sc_extract(x_hbm, out_hbm, b0, b1, vals_v, s0, s1, *, row0, n_per_w, nc,
                chan):
    wid = lax.axis_index("s") * nc + lax.axis_index("c")
    base = row0 + wid * n_per_w
    bufs = (b0, b1)
    sems = (s0, s1)
    n_chunks = n_per_w // _SC_CHUNK

    def _dma(k):
        return pltpu.make_async_copy(
            x_hbm.at[pl.ds(base + k * _SC_CHUNK, _SC_CHUNK), :],
            bufs[k % 2], sems[k % 2])

    lane = lax.broadcasted_iota(jnp.int32, (16,), 0)
    idx_chan = jnp.full((16,), chan, jnp.int32)

    _dma(0).start()
    for k in range(n_chunks):
        if k + 1 < n_chunks:
            _dma(k + 1).start()
        _dma(k).wait()
        buf = bufs[k % 2]

        def _grp(g, _, buf=buf, k=k):
            r0 = pl.multiple_of(g * 16, 16)
            acc = jnp.zeros((16,), jnp.float32)
            for j in range(16):
                v = buf[r0 + j, pl.ds(0, 16)]
                b = v.at[idx_chan].get(mode="promise_in_bounds")
                acc = jnp.where(lane == j, b, acc)
            vals_v[pl.ds(k * _SC_CHUNK + r0, 16)] = acc
            return 0

        lax.fori_loop(0, _SC_CHUNK // 16, _grp, 0, unroll=False)
    pltpu.sync_copy(vals_v, out_hbm.at[pl.ds(wid * n_per_w, n_per_w)])


def _sc_gather_channel(x2d, row0, n_out, chan):
    C = x2d.shape[1]
    info = plsc.get_sparse_core_info()
    nc, ns = info.num_cores, info.num_subcores
    n_per_w = n_out // (nc * ns)
    mesh = plsc.VectorSubcoreMesh(core_axis_name="c", subcore_axis_name="s")
    return pl.kernel(
        functools.partial(_sc_extract, row0=row0, n_per_w=n_per_w, nc=nc,
                          chan=chan),
        mesh=mesh,
        out_type=jax.ShapeDtypeStruct((n_out,), jnp.float32),
        scratch_types=[
            pltpu.VMEM((_SC_CHUNK, C), jnp.float32),
            pltpu.VMEM((_SC_CHUNK, C), jnp.float32),
            pltpu.VMEM((n_per_w,), jnp.float32),
            pltpu.SemaphoreType.DMA,
            pltpu.SemaphoreType.DMA,
        ],
    )(x2d)


def _tc_c_body(t_ref, an_ref, p_ref, sa_ref, out_ref, *, A, H, W, T_total,
               T_per_b, n_elems, row0, n_rows, n_lanes):
    pred = p_ref[:, :]

    hit, lin = _match_targets(t_ref, an_ref, A, H, W, T_per_b)
    v = lin - row0
    r_u = jax.lax.div(v, n_lanes)
    l_u = jax.lax.rem(v, n_lanes)

    row_iota = jax.lax.broadcasted_iota(jnp.int32, (n_rows, T_total), 0)
    lane_iota = jax.lax.broadcasted_iota(jnp.int32, (n_lanes, T_total), 0)
    rh = (row_iota == r_u).astype(jnp.float32) * hit
    lh = (lane_iota == l_u).astype(jnp.float32)
    cnt = jax.lax.dot_general(rh, lh, (((1,), (1,)), ((), ())),
                              preferred_element_type=jnp.float32)
    gt = cnt > 0.0

    s_sum = _bce_sum(pred, gt)
    out_ref[0, 0] = (s_sum + sa_ref[0, 0]) / float(n_elems)


def kernel(output, anchors, targets):
    B, A, H, W, C = output.shape
    T = targets.shape[1]
    n_elems = B * A * H * W
    n_planes = B * A
    x4d = output.reshape(n_planes, H, W, C)
    x2d = output.reshape(n_elems, C)
    tt = targets.reshape(B * T, 5).T

    row0 = _NA_PLANES * H * W
    n_b = n_elems - row0

    s_a = pl.pallas_call(
        functools.partial(_tc_a_body, A=A, H=H, W=W, T_total=B * T,
                          T_per_b=T),
        grid=(_NA_PLANES // _BS,),
        in_specs=[
            pl.BlockSpec((5, B * T), lambda i: (0, 0)),
            pl.BlockSpec((A, 2), lambda i: (0, 0)),
            pl.BlockSpec((_BS, H, W, C), lambda i: (i, 0, 0, 0)),
        ],
        out_specs=pl.BlockSpec(memory_space=pltpu.SMEM),
        out_shape=jax.ShapeDtypeStruct((1, 1), jnp.float32),
    )(tt, anchors, x4d)

    pred_b = _sc_gather_channel(x2d, row0, n_b, 4).reshape(n_b // 128, 128)

    out = pl.pallas_call(
        functools.partial(_tc_c_body, A=A, H=H, W=W, T_total=B * T,
                          T_per_b=T, n_elems=n_elems, row0=row0,
                          n_rows=n_b // 128, n_lanes=128),
        in_specs=[
            pl.BlockSpec((5, B * T), lambda: (0, 0)),
            pl.BlockSpec((A, 2), lambda: (0, 0)),
            pl.BlockSpec((n_b // 128, 128), lambda: (0, 0)),
            pl.BlockSpec(memory_space=pltpu.SMEM),
        ],
        out_specs=pl.BlockSpec(memory_space=pltpu.SMEM),
        out_shape=jax.ShapeDtypeStruct((1, 1), jnp.float32),
    )(tt, anchors, pred_b, s_a)
    return out[0, 0]


# speedup vs baseline: 2.1990x; 1.7985x over previous
import functools

import jax
import jax.numpy as jnp
from jax.experimental import pallas as pl
from jax.experimental.pallas import tpu as pltpu

_THRESHOLD = 0.5
_BS = 8
_NSTREAM = 2


def _match_targets(t_ref, an_ref, A, H, W, T_per_b):
    tx = t_ref[1:2, :]
    ty = t_ref[2:3, :]
    tw = t_ref[3:4, :] * float(W)
    th = t_ref[4:5, :] * float(H)
    area_t = tw * th

    best_iou = None
    best_a = jnp.zeros_like(tx, dtype=jnp.int32)
    for k in range(A):
        aw = an_ref[k:k + 1, 0:1]
        ah = an_ref[k:k + 1, 1:2]
        inter = jnp.minimum(aw, tw) * jnp.minimum(ah, th)
        iou = inter / (aw * ah + area_t - inter)
        if k == 0:
            best_iou = iou
        else:
            upd = iou > best_iou
            best_a = jnp.where(upd, k, best_a)
            best_iou = jnp.where(upd, iou, best_iou)

    t_i = (tx * float(W)).astype(jnp.int32)
    t_j = (ty * float(H)).astype(jnp.int32)
    t_b = jax.lax.broadcasted_iota(jnp.int32, tx.shape, 1) // T_per_b
    hit = (best_iou > _THRESHOLD).astype(jnp.float32)
    lin = ((t_b * A + best_a) * H + t_j) * W + t_i
    return hit, lin


def _body(t_ref, an_ref, xa_ref, xb_ref, out_ref, *, A, H, W, C, T_total,
          T_per_b, n_elems, n_steps_per_stream):
    i = pl.program_id(0)
    n = pl.num_programs(0)
    rows = _BS * H * W
    n_grp = rows // 128

    hit, lin = _match_targets(t_ref, an_ref, A, H, W, T_per_b)

    qi = jax.lax.broadcasted_iota(jnp.int32, (rows, 128), 0)
    li = jax.lax.broadcasted_iota(jnp.int32, (rows, 128), 1)
    lane_sel = (qi % 128 == li).astype(jnp.float32)
    qi2 = jax.lax.broadcasted_iota(jnp.int32, (rows, n_grp), 0)
    gi = jax.lax.broadcasted_iota(jnp.int32, (rows, n_grp), 1)
    cmat = (qi2 // 128 == gi).astype(jnp.float32)
    onesC = jnp.ones((C, 128), jnp.float32)
    lane4 = (jax.lax.broadcasted_iota(jnp.int32, (1, C), 1) == 4
             ).astype(jnp.float32)
    li_t = jax.lax.broadcasted_iota(jnp.int32, (128, T_total), 0)
    ai_t = jax.lax.broadcasted_iota(jnp.int32, (n_grp, T_total), 0)

    s_sum = jnp.float32(0.0)
    for stream, x_ref in enumerate((xa_ref, xb_ref)):
        base = (stream * n_steps_per_stream + i) * rows

        m = x_ref[...].reshape(rows, C)
        v0 = m * lane4
        r = jax.lax.dot_general(v0, onesC, (((1,), (0,)), ((), ())),
                                preferred_element_type=jnp.float32)
        z = r * lane_sel
        pred = jax.lax.dot_general(z, cmat, (((0,), (0,)), ((), ())),
                                   preferred_element_type=jnp.float32)

        v = lin - base
        a_u = jax.lax.div(v, 128)
        l_u = jax.lax.rem(v, 128)
        lh = (li_t == l_u).astype(jnp.float32) * hit
        ahot = (ai_t == a_u).astype(jnp.float32)
        cnt = jax.lax.dot_general(lh, ahot, (((1,), (1,)), ((), ())),
                                  preferred_element_type=jnp.float32)
        gt = cnt > 0.0

        log_p = jnp.maximum(jnp.log(pred), -100.0)
        log_1p = jnp.maximum(jnp.log(1.0 - pred), -100.0)
        s_sum = s_sum + jnp.sum(jnp.where(gt, -log_p, -log_1p))

    acc = jnp.where(i == 0, 0.0, out_ref[0, 0]) + s_sum
    out_ref[0, 0] = jnp.where(i == n - 1, acc / float(n_elems), acc)


def kernel(output, anchors, targets):
    B, A, H, W, C = output.shape
    T = targets.shape[1]
    n_elems = B * A * H * W
    n_planes = B * A
    x4d = output.reshape(n_planes, H, W, C)
    tt = targets.reshape(B * T, 5).T
    n_steps = n_planes // (_BS * _NSTREAM)

    out = pl.pallas_call(
        functools.partial(_body, A=A, H=H, W=W, C=C, T_total=B * T,
                          T_per_b=T, n_elems=n_elems,
                          n_steps_per_stream=n_steps),
        grid=(n_steps,),
        in_specs=[
            pl.BlockSpec((5, B * T), lambda i: (0, 0)),
            pl.BlockSpec((A, 2), lambda i: (0, 0)),
            pl.BlockSpec((_BS, H, W, C), lambda i: (i, 0, 0, 0)),
            pl.BlockSpec((_BS, H, W, C), lambda i, n=n_steps: (i + n, 0, 0, 0)),
        ],
        out_specs=pl.BlockSpec(memory_space=pltpu.SMEM),
        out_shape=jax.ShapeDtypeStruct((1, 1), jnp.float32),
    )(tt, anchors, x4d, x4d)
    return out[0, 0]
